# Initial kernel scaffold; baseline (speedup 1.0000x reference)
import jax
import jax.numpy as jnp
from jax.experimental import pallas as pl


def kernel(x, edge_index, W1, b1, W2, b2, W_halt, b_halt):
    raise NotImplementedError("write your pallas kernel here")


# speedup vs baseline: 3.9042x; 3.9042x over previous
import functools

import jax
import jax.numpy as jnp
from jax import lax
from jax.experimental import pallas as pl
from jax.experimental.pallas import tpu as pltpu
from jax.experimental.pallas import tpu_sc as plsc

N = 50000
E = 800000
D = 256
F = 64
NQ = 8
QW = F // NQ
NITER = 10

NC = 2
NS = 16
NPH = NQ // NC
CHUNK = 128
CPT = 392
EPT = CPT * CHUNK
EPAD = NS * EPT
NPAD = 50048
DW = 8
ZR = NPAD // NS
OR = 3128
ORL = N - (NS - 1) * OR

TCB = 1000
TCG = N // TCB


def _copy_out(sh, hbm, f, s):
    off = s * OR

    @pl.when(s < NS - 1)
    def _():
        pltpu.sync_copy(sh.at[pl.ds(off, OR)], hbm.at[f, pl.ds(off, OR)])

    @pl.when(s == NS - 1)
    def _():
        pltpu.sync_copy(sh.at[pl.ds(off, ORL)], hbm.at[f, pl.ds(off, ORL)])


def _spmm_body(q_ref, rowi_ref, coli_ref, zeros_ref, acc_ref,
               rowi_v, coli_v, buf, acc_sh, sem):
    c = lax.axis_index("c")
    s = lax.axis_index("s")
    pltpu.sync_copy(coli_ref.at[s], coli_v)
    for p in range(NPH):
        f = NC * p + c
        pltpu.sync_copy(zeros_ref.at[pl.ds(s * ZR, ZR)],
                        acc_sh.at[pl.ds(s * ZR, ZR)])
        pltpu.sync_copy(rowi_ref.at[f * NS + s], rowi_v)
        plsc.subcore_barrier()

        def chunk(j, carry):
            pltpu.async_copy(q_ref.at[rowi_v.at[j]], buf, sem).wait()
            pltpu.sync_copy(buf, acc_sh.at[coli_v.at[j]], add=True)
            return carry

        lax.fori_loop(0, CPT, chunk, 0)
        plsc.subcore_barrier()
        _copy_out(acc_sh, acc_ref, f, s)
        plsc.subcore_barrier()


@functools.cache
def _get_spmm():
    return pl.kernel(
        _spmm_body,
        out_type=jax.ShapeDtypeStruct((NQ, N, QW), jnp.float32),
        mesh=plsc.VectorSubcoreMesh(core_axis_name="c", subcore_axis_name="s",
                                    num_cores=NC, num_subcores=NS),
        scratch_types=[
            pltpu.VMEM((CPT, CHUNK), jnp.int32),
            pltpu.VMEM((CPT, CHUNK), jnp.int32),
            pltpu.VMEM((CHUNK, QW), jnp.float32),
            pltpu.VMEM_SHARED((NPAD, QW), jnp.float32),
            pltpu.SemaphoreType.DMA,
        ],
        compiler_params=pltpu.CompilerParams(use_tc_tiling_on_sc=False),
    )


def _deg_body(coli_ref, zeros_ref, ones_ref, deg_ref, coli_v, ones_v, deg_sh):
    c = lax.axis_index("c")
    s = lax.axis_index("s")
    pltpu.sync_copy(ones_ref, ones_v)
    pltpu.sync_copy(zeros_ref.at[pl.ds(s * ZR, ZR)], deg_sh.at[pl.ds(s * ZR, ZR)])
    pltpu.sync_copy(coli_ref.at[c * NS + s], coli_v)
    plsc.subcore_barrier()

    def chunk(j, carry):
        pltpu.sync_copy(ones_v, deg_sh.at[coli_v.at[j]], add=True)
        return carry

    lax.fori_loop(0, CPT // 2, chunk, 0)
    plsc.subcore_barrier()
    _copy_out(deg_sh, deg_ref, c, s)


@functools.cache
def _get_deg():
    return pl.kernel(
        _deg_body,
        out_type=jax.ShapeDtypeStruct((NC, N, DW), jnp.float32),
        mesh=plsc.VectorSubcoreMesh(core_axis_name="c", subcore_axis_name="s",
                                    num_cores=NC, num_subcores=NS),
        scratch_types=[
            pltpu.VMEM((CPT // 2, CHUNK), jnp.int32),
            pltpu.VMEM((CHUNK, DW), jnp.float32),
            pltpu.VMEM_SHARED((NPAD, DW), jnp.float32),
        ],
        compiler_params=pltpu.CompilerParams(use_tc_tiling_on_sc=False),
    )


def _mlp_body(x_ref, w1_ref, b1_ref, w2_ref, b2_ref, o_ref):
    h = jnp.dot(x_ref[...], w1_ref[...], preferred_element_type=jnp.float32)
    h = jnp.maximum(h + b1_ref[...], 0.0)
    o_ref[...] = jnp.dot(h, w2_ref[...], preferred_element_type=jnp.float32) + b2_ref[...]


def _prep_body(degp_ref, lp_ref, dinv_ref, q0_ref, prop0_ref):
    deg = degp_ref[0][:, 0:1] + degp_ref[1][:, 0:1] + 1.0
    dv = lax.rsqrt(deg)
    dinv_ref[...] = dv
    lp = lp_ref[...]
    q0_ref[...] = dv * lp
    prop0_ref[...] = lp


def _halt_body(acc_ref, q_ref, prop_ref, xacc_ref, steps_ref, sumh_ref,
               cont_ref, dinv_ref, wh_ref, bh_ref,
               propn_ref, xaccn_ref, qn_ref, stepsn_ref, sumhn_ref, contn_ref):
    dv = dinv_ref[...]
    pn = dv * (acc_ref[...] + q_ref[...])
    z = jnp.sum(pn * wh_ref[...], axis=1, keepdims=True) + bh_ref[...]
    h = 1.0 / (1.0 + jnp.exp(-z))
    st = steps_ref[...]
    sh = sumh_ref[...]
    co = cont_ref[...]
    pm = jnp.logical_and((sh + h) < 0.99, co > 0.5)
    pf = pm.astype(jnp.float32)
    st_n = st + pf
    sh_n = sh + pf * h
    cond = jnp.logical_and(pm, st_n < float(NITER))
    p = jnp.where(cond, sh_n, 1.0 - sh_n)
    xaccn_ref[...] = xacc_ref[...] + (p * pn + (1.0 - p) * prop_ref[...]) * co
    propn_ref[...] = pn
    qn_ref[...] = dv * pn
    stepsn_ref[...] = st_n
    sumhn_ref[...] = sh_n
    contn_ref[...] = pf


def _epi_body(xacc_ref, steps_ref, sumh_ref, logp_ref, rem_ref):
    st = steps_ref[...]
    o = xacc_ref[...] / st
    m = o.max(axis=1, keepdims=True)
    lse = m + jnp.log(jnp.sum(jnp.exp(o - m), axis=1, keepdims=True))
    logp_ref[...] = o - lse
    rem_ref[...] = 1.0 - sumh_ref[...]


def _nb(w):
    return pl.BlockSpec((TCB, w), lambda i: (i, 0))


def _const(*shape):
    nd = len(shape)
    return pl.BlockSpec(shape, lambda i, _n=nd: (0,) * _n)


def kernel(x, edge_index, W1, b1, W2, b2, W_halt, b_halt):
    f32 = jnp.float32
    row = edge_index[0]
    col = edge_index[1]
    pad = EPAD - E
    rowp = jnp.concatenate([row, jnp.zeros((pad,), jnp.int32)])
    colp = jnp.concatenate([col, jnp.full((pad,), N, jnp.int32)])
    rowi = jnp.concatenate([rowp * NQ + f for f in range(NQ)]).reshape(
        NQ * NS, CPT, CHUNK)
    coli = colp.reshape(NS, CPT, CHUNK)
    coli_deg = (colp.reshape(NS, NC, CPT // 2, CHUNK)
                .transpose(1, 0, 2, 3).reshape(NC * NS, CPT // 2, CHUNK))
    zeros_sp = jnp.zeros((NPAD, QW), f32)
    zeros_deg = jnp.zeros((NPAD, DW), f32)
    ones_deg = jnp.ones((CHUNK, DW), f32)

    local_preds = pl.pallas_call(
        _mlp_body,
        grid=(TCG,),
        in_specs=[_nb(D), _const(D, F), _const(1, F), _const(F, F), _const(1, F)],
        out_specs=_nb(F),
        out_shape=jax.ShapeDtypeStruct((N, F), f32),
    )(x, W1, b1.reshape(1, F), W2, b2.reshape(1, F))

    degp = _get_deg()(coli_deg, zeros_deg, ones_deg)

    dinv, q, prop = pl.pallas_call(
        _prep_body,
        grid=(TCG,),
        in_specs=[pl.BlockSpec((NC, TCB, DW), lambda i: (0, i, 0)), _nb(F)],
        out_specs=[_nb(1), _nb(F), _nb(F)],
        out_shape=[
            jax.ShapeDtypeStruct((N, 1), f32),
            jax.ShapeDtypeStruct((N, F), f32),
            jax.ShapeDtypeStruct((N, F), f32),
        ],
    )(degp, local_preds)

    xacc = jnp.zeros((N, F), f32)
    steps = jnp.ones((N, 1), f32)
    sum_h = jnp.zeros((N, 1), f32)
    cont = jnp.ones((N, 1), f32)
    wh = W_halt.reshape(1, F)
    bh = b_halt.reshape(1, 1)

    halt_call = pl.pallas_call(
        _halt_body,
        grid=(TCG,),
        in_specs=[_nb(F), _nb(F), _nb(F), _nb(F), _nb(1), _nb(1), _nb(1),
                  _nb(1), _const(1, F), _const(1, 1)],
        out_specs=[_nb(F), _nb(F), _nb(F), _nb(1), _nb(1), _nb(1)],
        out_shape=[
            jax.ShapeDtypeStruct((N, F), f32),
            jax.ShapeDtypeStruct((N, F), f32),
            jax.ShapeDtypeStruct((N, F), f32),
            jax.ShapeDtypeStruct((N, 1), f32),
            jax.ShapeDtypeStruct((N, 1), f32),
            jax.ShapeDtypeStruct((N, 1), f32),
        ],
    )

    for _ in range(NITER):
        acc8 = _get_spmm()(q.reshape(N * NQ, QW), rowi, coli, zeros_sp)
        acc = jnp.swapaxes(acc8, 0, 1).reshape(N, F)
        prop, xacc, q, steps, sum_h, cont = halt_call(
            acc, q, prop, xacc, steps, sum_h, cont, dinv, wh, bh)

    logp, rem = pl.pallas_call(
        _epi_body,
        grid=(TCG,),
        in_specs=[_nb(F), _nb(1), _nb(1)],
        out_specs=[_nb(F), _nb(1)],
        out_shape=[
            jax.ShapeDtypeStruct((N, F), f32),
            jax.ShapeDtypeStruct((N, 1), f32),
        ],
    )(xacc, steps, sum_h)

    return (logp, steps.reshape(N), rem.reshape(N))


# speedup vs baseline: 6.7435x; 1.7272x over previous
import functools

import jax
import jax.numpy as jnp
from jax import lax
from jax.experimental import pallas as pl
from jax.experimental.pallas import tpu as pltpu
from jax.experimental.pallas import tpu_sc as plsc

N = 50000
E = 800000
D = 256
F = 64
NQ = 8
QW = F // NQ
NITER = 10

NC = 2
NS = 16
NPH = NQ // NC
CHUNK = 128
CPT = 392
EPT = CPT * CHUNK
EPAD = NS * EPT
NPAD = 50048
DW = 8
ZR = NPAD // NS
OR = 3128
ORL = N - (NS - 1) * OR
CROWS = NPAD * QW // 128
CCR = CROWS // NS
QROWS = N * F // 128
NBUF = 4
NBK = NBUF // 2

TCB = 1000
TCG = N // TCB


def _copy_out(sh, hbm, f, s):
    off = s * OR

    @pl.when(s < NS - 1)
    def _():
        pltpu.sync_copy(sh.at[pl.ds(off, OR)], hbm.at[f, pl.ds(off, OR)])

    @pl.when(s == NS - 1)
    def _():
        pltpu.sync_copy(sh.at[pl.ds(off, ORL)], hbm.at[f, pl.ds(off, ORL)])


def _spmm_body(q_ref, rowi_ref, coli_ref, zeros_ref, acc_ref,
               rowi_v, coli_v, bufs, acc_sh, sem):
    sem_a = sem
    sem_b = sem
    c = lax.axis_index("c")
    s = lax.axis_index("s")
    pltpu.sync_copy(coli_ref.at[s], coli_v)

    def gather(j, b, sem):
        pltpu.async_copy(q_ref.at[rowi_v.at[j]], bufs.at[b], sem)

    def drain(j, b, sem):
        pltpu.make_async_copy(q_ref.at[rowi_v.at[j]], bufs.at[b], sem).wait()

    def scatter(j, b):
        pltpu.sync_copy(bufs.at[b], acc_sh.at[coli_v.at[j]], add=True)

    for p in range(NPH):
        f = NC * p + c
        pltpu.sync_copy(zeros_ref.at[pl.ds(s * ZR, ZR)],
                        acc_sh.at[pl.ds(s * ZR, ZR)])
        pltpu.sync_copy(rowi_ref.at[f * NS + s], rowi_v)
        plsc.subcore_barrier()
        for b in range(NBK):
            gather(b, b, sem_a)

        def group(g, carry):
            j0 = g * 2 * NBK
            for b in range(NBK):
                gather(j0 + NBK + b, NBK + b, sem_b)
            for b in range(NBK):
                drain(j0 + b, b, sem_a)
            for b in range(NBK):
                scatter(j0 + b, b)
            for b in range(NBK):
                nj = j0 + 2 * NBK + b

                @pl.when(nj < CPT)
                def _():
                    gather(nj, b, sem_a)
            for b in range(NBK):
                drain(j0 + NBK + b, NBK + b, sem_b)
            for b in range(NBK):
                scatter(j0 + NBK + b, NBK + b)
            return carry

        lax.fori_loop(0, CPT // (2 * NBK), group, 0)
        plsc.subcore_barrier()
        _copy_out(acc_sh, acc_ref, f, s)
        plsc.subcore_barrier()


@functools.cache
def _get_spmm():
    return pl.kernel(
        _spmm_body,
        out_type=jax.ShapeDtypeStruct((NQ, N, QW), jnp.float32),
        mesh=plsc.VectorSubcoreMesh(core_axis_name="c", subcore_axis_name="s",
                                    num_cores=NC, num_subcores=NS),
        scratch_types=[
            pltpu.VMEM((CPT, CHUNK), jnp.int32),
            pltpu.VMEM((CPT, CHUNK), jnp.int32),
            pltpu.VMEM((NBUF, CHUNK, QW), jnp.float32),
            pltpu.VMEM_SHARED((NPAD, QW), jnp.float32),
            pltpu.SemaphoreType.DMA,
        ],
        compiler_params=pltpu.CompilerParams(use_tc_tiling_on_sc=False),
    )


def _deg_body(coli_ref, zeros_ref, ones_ref, deg_ref, coli_v, ones_v, deg_sh):
    c = lax.axis_index("c")
    s = lax.axis_index("s")
    pltpu.sync_copy(ones_ref, ones_v)
    pltpu.sync_copy(zeros_ref.at[pl.ds(s * ZR, ZR)], deg_sh.at[pl.ds(s * ZR, ZR)])
    pltpu.sync_copy(coli_ref.at[c * NS + s], coli_v)
    plsc.subcore_barrier()

    def chunk(j, carry):
        pltpu.sync_copy(ones_v, deg_sh.at[coli_v.at[j]], add=True)
        return carry

    lax.fori_loop(0, CPT // 2, chunk, 0)
    plsc.subcore_barrier()
    _copy_out(deg_sh, deg_ref, c, s)


@functools.cache
def _get_deg():
    return pl.kernel(
        _deg_body,
        out_type=jax.ShapeDtypeStruct((NC, N, DW), jnp.float32),
        mesh=plsc.VectorSubcoreMesh(core_axis_name="c", subcore_axis_name="s",
                                    num_cores=NC, num_subcores=NS),
        scratch_types=[
            pltpu.VMEM((CPT // 2, CHUNK), jnp.int32),
            pltpu.VMEM((CHUNK, DW), jnp.float32),
            pltpu.VMEM_SHARED((NPAD, DW), jnp.float32),
        ],
        compiler_params=pltpu.CompilerParams(use_tc_tiling_on_sc=False),
    )


def _mlp_body(x_ref, w1_ref, b1_ref, w2_ref, b2_ref, o_ref):
    h = jnp.dot(x_ref[...], w1_ref[...], preferred_element_type=jnp.float32)
    h = jnp.maximum(h + b1_ref[...], 0.0)
    o_ref[...] = jnp.dot(h, w2_ref[...], preferred_element_type=jnp.float32) + b2_ref[...]


def _prep_body(degp_ref, lp_ref, dinv_ref, q0_ref, prop0_ref):
    deg = degp_ref[0][:, 0:1] + degp_ref[1][:, 0:1] + 1.0
    dv = lax.rsqrt(deg)
    dinv_ref[...] = dv
    lp = lp_ref[...]
    q0_ref[...] = dv * lp
    prop0_ref[...] = lp


def _halt_body(acc_ref, q_ref, prop_ref, xacc_ref, steps_ref, sumh_ref,
               cont_ref, dinv_ref, wh_ref, bh_ref,
               propn_ref, xaccn_ref, qn_ref, stepsn_ref, sumhn_ref, contn_ref):
    dv = dinv_ref[...]
    pn = dv * (acc_ref[...] + q_ref[...])
    z = jnp.sum(pn * wh_ref[...], axis=1, keepdims=True) + bh_ref[...]
    h = 1.0 / (1.0 + jnp.exp(-z))
    st = steps_ref[...]
    sh = sumh_ref[...]
    co = cont_ref[...]
    pm = jnp.logical_and((sh + h) < 0.99, co > 0.5)
    pf = pm.astype(jnp.float32)
    st_n = st + pf
    sh_n = sh + pf * h
    cond = jnp.logical_and(pm, st_n < float(NITER))
    p = jnp.where(cond, sh_n, 1.0 - sh_n)
    xaccn_ref[...] = xacc_ref[...] + (p * pn + (1.0 - p) * prop_ref[...]) * co
    propn_ref[...] = pn
    qn_ref[...] = dv * pn
    stepsn_ref[...] = st_n
    sumhn_ref[...] = sh_n
    contn_ref[...] = pf


def _epi_body(xacc_ref, steps_ref, sumh_ref, logp_ref, rem_ref):
    st = steps_ref[...]
    o = xacc_ref[...] / st
    m = o.max(axis=1, keepdims=True)
    lse = m + jnp.log(jnp.sum(jnp.exp(o - m), axis=1, keepdims=True))
    logp_ref[...] = o - lse
    rem_ref[...] = 1.0 - sumh_ref[...]


def _nb(w):
    return pl.BlockSpec((TCB, w), lambda i: (i, 0))


def _const(*shape):
    nd = len(shape)
    return pl.BlockSpec(shape, lambda i, _n=nd: (0,) * _n)


def kernel(x, edge_index, W1, b1, W2, b2, W_halt, b_halt):
    f32 = jnp.float32
    row = edge_index[0]
    col = edge_index[1]
    pad = EPAD - E
    rowp = jnp.concatenate([row, jnp.zeros((pad,), jnp.int32)])
    colp = jnp.concatenate([col, jnp.full((pad,), N, jnp.int32)])
    rowi = jnp.concatenate([rowp * NQ + f for f in range(NQ)]).reshape(
        NQ * NS, CPT, CHUNK)
    coli = colp.reshape(NS, CPT, CHUNK)
    coli_deg = (colp.reshape(NS, NC, CPT // 2, CHUNK)
                .transpose(1, 0, 2, 3).reshape(NC * NS, CPT // 2, CHUNK))
    zeros_sp = jnp.zeros((NPAD, QW), f32)
    zeros_deg = jnp.zeros((NPAD, DW), f32)
    ones_deg = jnp.ones((CHUNK, DW), f32)

    local_preds = pl.pallas_call(
        _mlp_body,
        grid=(TCG,),
        in_specs=[_nb(D), _const(D, F), _const(1, F), _const(F, F), _const(1, F)],
        out_specs=_nb(F),
        out_shape=jax.ShapeDtypeStruct((N, F), f32),
    )(x, W1, b1.reshape(1, F), W2, b2.reshape(1, F))

    degp = _get_deg()(coli_deg, zeros_deg, ones_deg)

    dinv, q, prop = pl.pallas_call(
        _prep_body,
        grid=(TCG,),
        in_specs=[pl.BlockSpec((NC, TCB, DW), lambda i: (0, i, 0)), _nb(F)],
        out_specs=[_nb(1), _nb(F), _nb(F)],
        out_shape=[
            jax.ShapeDtypeStruct((N, 1), f32),
            jax.ShapeDtypeStruct((N, F), f32),
            jax.ShapeDtypeStruct((N, F), f32),
        ],
    )(degp, local_preds)

    xacc = jnp.zeros((N, F), f32)
    steps = jnp.ones((N, 1), f32)
    sum_h = jnp.zeros((N, 1), f32)
    cont = jnp.ones((N, 1), f32)
    wh = W_halt.reshape(1, F)
    bh = b_halt.reshape(1, 1)

    halt_call = pl.pallas_call(
        _halt_body,
        grid=(TCG,),
        in_specs=[_nb(F), _nb(F), _nb(F), _nb(F), _nb(1), _nb(1), _nb(1),
                  _nb(1), _const(1, F), _const(1, 1)],
        out_specs=[_nb(F), _nb(F), _nb(F), _nb(1), _nb(1), _nb(1)],
        out_shape=[
            jax.ShapeDtypeStruct((N, F), f32),
            jax.ShapeDtypeStruct((N, F), f32),
            jax.ShapeDtypeStruct((N, F), f32),
            jax.ShapeDtypeStruct((N, 1), f32),
            jax.ShapeDtypeStruct((N, 1), f32),
            jax.ShapeDtypeStruct((N, 1), f32),
        ],
    )

    for _ in range(NITER):
        acc8 = _get_spmm()(q.reshape(N * NQ, QW), rowi, coli, zeros_sp)
        acc = jnp.swapaxes(acc8, 0, 1).reshape(N, F)
        prop, xacc, q, steps, sum_h, cont = halt_call(
            acc, q, prop, xacc, steps, sum_h, cont, dinv, wh, bh)

    logp, rem = pl.pallas_call(
        _epi_body,
        grid=(TCG,),
        in_specs=[_nb(F), _nb(1), _nb(1)],
        out_specs=[_nb(F), _nb(1)],
        out_shape=[
            jax.ShapeDtypeStruct((N, F), f32),
            jax.ShapeDtypeStruct((N, 1), f32),
        ],
    )(xacc, steps, sum_h)

    return (logp, steps.reshape(N), rem.reshape(N))
